# baseline (speedup 1.0000x reference)
import jax
import jax.numpy as jnp
from jax import lax
from jax.experimental import pallas as pl
from jax.experimental.pallas import tpu as pltpu
from jax.experimental.pallas import tpu_sc as plsc

B, L, H = 16, 4096, 256
NC, NS = 2, 16
NW = NC * NS
WPB = NW // B
TOK_PER_W = L // WPB
TT = 128
NT = TOK_PER_W // TT
RC = 64
HV = H // 16


def _sget(ref, idx):
    return ref[pl.ds(idx, 16)][0]


def _lower_bound(seg_v, t):
    def step(_, lh):
        lo, hi = lh
        mid = (lo + hi) // 2
        v = _sget(seg_v, mid)
        active = lo < hi
        lt = (v < t) & active
        ge = jnp.logical_not(v < t) & active
        return jnp.where(lt, mid + 1, lo), jnp.where(ge, mid, hi)
    lo, _ = lax.fori_loop(0, 12, step, (jnp.int32(0), jnp.int32(L)))
    return lo


def _body(x_hbm, seg_hbm, out_hbm, seg_v, in0, in1, ob0, ob1,
          sem_i0, sem_i1, sem_o0, sem_o1):
    c = lax.axis_index("c")
    s = lax.axis_index("s")
    wid = s * NC + c
    b = wid // WPB
    t_base = (wid % WPB) * TOK_PER_W

    inbufs = (in0, in1)
    outbufs = (ob0, ob1)
    sems_i = (sem_i0, sem_i1)
    sems_o = (sem_o0, sem_o1)
    zv = jnp.zeros((16,), jnp.float32)

    pltpu.sync_copy(seg_hbm.at[b], seg_v.at[pl.ds(0, L)])
    seg_v[pl.ds(L, 16)] = jnp.full((16,), L, jnp.int32)
    r_start = _lower_bound(seg_v, t_base)

    def in_dma(ci, r0, r1, buf, sem):
        start = r0 + ci * RC
        d = jnp.minimum(start, L - RC)
        return pltpu.make_async_copy(x_hbm.at[b, pl.ds(d * H, RC * H)], buf, sem)

    def out_dma(k, buf, sem):
        t_lo = t_base + k * TT
        return pltpu.make_async_copy(buf, out_hbm.at[b, pl.ds(t_lo * H, TT * H)], sem)

    def run_tile(kq, q, r0):
        k = 2 * kq + q
        t_lo = t_base + k * TT
        r1 = _lower_bound(seg_v, t_lo + TT)
        outbuf = outbufs[q]

        @pl.when(kq >= 1)
        def _():
            out_dma(k - 2, outbuf, sems_o[q]).wait()

        nch = (r1 - r0 + RC - 1) // RC

        @pl.when(nch > 0)
        def _():
            in_dma(0, r0, r1, inbufs[0], sems_i[0]).start()

        def zero_tokens(t0, t1):
            def zbody(t, carry):
                base = t * H
                for h in range(HV):
                    outbuf[pl.ds(base + 16 * h, 16)] = zv
                return carry
            lax.fori_loop(t0, t1, zbody, 0)

        def chunk_pair(cj, carry):
            for p in (0, 1):
                ci = 2 * cj + p

                @pl.when(ci < nch)
                def _():
                    in_dma(ci, r0, r1, inbufs[p], sems_i[p]).wait()

                    @pl.when(ci + 1 < nch)
                    def _():
                        in_dma(ci + 1, r0, r1, inbufs[1 - p], sems_i[1 - p]).start()

                start = r0 + ci * RC
                d = jnp.minimum(start, L - RC)
                off = start - d
                cnt = jnp.clip(r1 - start, 0, RC)
                inbuf = inbufs[p]

                def row_body(i, prev_s):
                    sgid = _sget(seg_v, start + i)
                    ii = off + i
                    nz = sgid + jnp.where(sgid != prev_s, 1, 0)
                    zero_tokens(prev_s + 1 - t_lo, nz - t_lo)
                    obase = (sgid - t_lo) * H
                    for h in range(HV):
                        xv = inbuf[pl.ds(ii * H + 16 * h, 16)]
                        plsc.addupdate(outbuf.at[pl.ds(obase + 16 * h, 16)], xv)
                    return sgid

                carry = lax.fori_loop(0, cnt, row_body, carry)
            return carry

        last = lax.fori_loop(0, (nch + 1) // 2, chunk_pair, t_lo - 1)
        zero_tokens(last + 1 - t_lo, TT)
        out_dma(k, outbuf, sems_o[q]).start()
        return r1

    def tile_pair(kq, r0):
        r0 = run_tile(kq, 0, r0)
        r0 = run_tile(kq, 1, r0)
        return r0

    lax.fori_loop(0, NT // 2, tile_pair, r_start)
    out_dma(NT - 2, outbufs[0], sems_o[0]).wait()
    out_dma(NT - 1, outbufs[1], sems_o[1]).wait()


@jax.jit
def kernel(sequence_output, wp_segment_ids):
    x = sequence_output.reshape(B, L * H)
    seg = wp_segment_ids.astype(jnp.int32)
    run = pl.kernel(
        _body,
        out_type=jax.ShapeDtypeStruct((B, L * H), jnp.float32),
        mesh=plsc.VectorSubcoreMesh(core_axis_name="c", subcore_axis_name="s"),
        scratch_types=[
            pltpu.VMEM((L + 16,), jnp.int32),
            pltpu.VMEM((RC * H,), jnp.float32),
            pltpu.VMEM((RC * H,), jnp.float32),
            pltpu.VMEM((TT * H,), jnp.float32),
            pltpu.VMEM((TT * H,), jnp.float32),
            pltpu.SemaphoreType.DMA,
            pltpu.SemaphoreType.DMA,
            pltpu.SemaphoreType.DMA,
            pltpu.SemaphoreType.DMA,
        ],
    )
    out = run(x, seg)
    return out.reshape(B, L, H)


# speedup vs baseline: 1.3850x; 1.3850x over previous
import jax
import jax.numpy as jnp
from jax import lax
from jax.experimental import pallas as pl
from jax.experimental.pallas import tpu as pltpu
from jax.experimental.pallas import tpu_sc as plsc

B, L, H = 16, 4096, 256
NC, NS = 2, 16
NW = NC * NS
WPB = NW // B
TOK_PER_W = L // WPB
TT = 128
NT = TOK_PER_W // TT
RC = 64
HV = H // 16


def _sget(ref, idx):
    return ref[pl.ds(idx, 16)][0]


def _lower_bound(seg_v, t):
    def step(_, lh):
        lo, hi = lh
        mid = (lo + hi) // 2
        v = _sget(seg_v, mid)
        active = lo < hi
        lt = (v < t) & active
        ge = jnp.logical_not(v < t) & active
        return jnp.where(lt, mid + 1, lo), jnp.where(ge, mid, hi)
    lo, _ = lax.fori_loop(0, 12, step, (jnp.int32(0), jnp.int32(L)))
    return lo


def _body(x_hbm, seg_hbm, out_hbm, seg_v, in0, in1, ob0, ob1,
          sem_i0, sem_i1, sem_o0, sem_o1):
    c = lax.axis_index("c")
    s = lax.axis_index("s")
    wid = s * NC + c
    b = wid // WPB
    t_base = (wid % WPB) * TOK_PER_W

    inbufs = (in0, in1)
    outbufs = (ob0, ob1)
    sems_i = (sem_i0, sem_i1)
    sems_o = (sem_o0, sem_o1)
    zv = jnp.zeros((16,), jnp.float32)

    pltpu.sync_copy(seg_hbm.at[b], seg_v.at[pl.ds(0, L)])
    seg_v[pl.ds(L, 16)] = jnp.full((16,), L, jnp.int32)
    r_start = _lower_bound(seg_v, t_base)

    def in_dma(ci, r0, r1, buf, sem):
        start = r0 + ci * RC
        d = jnp.minimum(start, L - RC)
        return pltpu.make_async_copy(x_hbm.at[b, pl.ds(d * H, RC * H)], buf, sem)

    def out_dma(k, buf, sem):
        t_lo = t_base + k * TT
        return pltpu.make_async_copy(buf, out_hbm.at[b, pl.ds(t_lo * H, TT * H)], sem)

    def run_tile(kq, q, r0):
        k = 2 * kq + q
        t_lo = t_base + k * TT
        r1 = _lower_bound(seg_v, t_lo + TT)
        outbuf = outbufs[q]

        @pl.when(kq >= 1)
        def _():
            out_dma(k - 2, outbuf, sems_o[q]).wait()

        nch = (r1 - r0 + RC - 1) // RC

        @pl.when(nch > 0)
        def _():
            in_dma(0, r0, r1, inbufs[0], sems_i[0]).start()

        def zero_tokens(t0, t1):
            def zbody(t, carry):
                base = t * H
                for h in range(HV):
                    outbuf[pl.ds(base + 16 * h, 16)] = zv
                return carry
            lax.fori_loop(t0, t1, zbody, 0)

        def chunk_pair(cj, carry):
            for p in (0, 1):
                ci = 2 * cj + p

                @pl.when(ci < nch)
                def _():
                    in_dma(ci, r0, r1, inbufs[p], sems_i[p]).wait()

                    @pl.when(ci + 1 < nch)
                    def _():
                        in_dma(ci + 1, r0, r1, inbufs[1 - p], sems_i[1 - p]).start()

                start = r0 + ci * RC
                d = jnp.minimum(start, L - RC)
                off = start - d
                cnt = jnp.clip(r1 - start, 0, RC)
                inbuf = inbufs[p]

                def row_body(i, rc):
                    prev_s = rc[0]
                    acc = rc[1:]
                    sgid = _sget(seg_v, start + i)
                    ii = off + i
                    same = sgid == prev_s
                    zero_tokens(prev_s + 1 - t_lo, sgid - t_lo)
                    new_acc = []
                    for h in range(HV):
                        xv = inbuf[pl.ds(ii * H + 16 * h, 16)]
                        new_acc.append(xv + jnp.where(same, acc[h], zv))
                    obase = (sgid - t_lo) * H
                    for h in range(HV):
                        outbuf[pl.ds(obase + 16 * h, 16)] = new_acc[h]
                    return (sgid,) + tuple(new_acc)

                carry = lax.fori_loop(0, cnt, row_body, carry)
            return carry

        init = (t_lo - 1,) + tuple(zv for _ in range(HV))
        last = lax.fori_loop(0, (nch + 1) // 2, chunk_pair, init)
        zero_tokens(last[0] + 1 - t_lo, TT)
        out_dma(k, outbuf, sems_o[q]).start()
        return r1

    def tile_pair(kq, r0):
        r0 = run_tile(kq, 0, r0)
        r0 = run_tile(kq, 1, r0)
        return r0

    lax.fori_loop(0, NT // 2, tile_pair, r_start)
    out_dma(NT - 2, outbufs[0], sems_o[0]).wait()
    out_dma(NT - 1, outbufs[1], sems_o[1]).wait()


@jax.jit
def kernel(sequence_output, wp_segment_ids):
    x = sequence_output.reshape(B, L * H)
    seg = wp_segment_ids.astype(jnp.int32)
    run = pl.kernel(
        _body,
        out_type=jax.ShapeDtypeStruct((B, L * H), jnp.float32),
        mesh=plsc.VectorSubcoreMesh(core_axis_name="c", subcore_axis_name="s"),
        scratch_types=[
            pltpu.VMEM((L + 16,), jnp.int32),
            pltpu.VMEM((RC * H,), jnp.float32),
            pltpu.VMEM((RC * H,), jnp.float32),
            pltpu.VMEM((TT * H,), jnp.float32),
            pltpu.VMEM((TT * H,), jnp.float32),
            pltpu.SemaphoreType.DMA,
            pltpu.SemaphoreType.DMA,
            pltpu.SemaphoreType.DMA,
            pltpu.SemaphoreType.DMA,
        ],
    )
    out = run(x, seg)
    return out.reshape(B, L, H)


# speedup vs baseline: 2.2996x; 1.6604x over previous
import jax
import jax.numpy as jnp
from jax import lax
from jax.experimental import pallas as pl
from jax.experimental.pallas import tpu as pltpu
from jax.experimental.pallas import tpu_sc as plsc

B, L, H = 16, 4096, 256
NC, NS = 2, 16
NW = NC * NS
WPB = NW // B
TOK_PER_W = L // WPB
TT = 128
NT = TOK_PER_W // TT
RC = 64
HV = H // 16


def _sget(ref, idx):
    return ref[pl.ds(idx, 16)][0]


def _lower_bound(seg_v, t):
    def step(_, lh):
        lo, hi = lh
        mid = (lo + hi) // 2
        v = _sget(seg_v, mid)
        active = lo < hi
        lt = (v < t) & active
        ge = jnp.logical_not(v < t) & active
        return jnp.where(lt, mid + 1, lo), jnp.where(ge, mid, hi)
    lo, _ = lax.fori_loop(0, 12, step, (jnp.int32(0), jnp.int32(L)))
    return lo


def _body(x_hbm, seg_hbm, out_hbm, seg_v, in0, in1, ob0, ob1,
          sem_i0, sem_i1, sem_o0, sem_o1):
    c = lax.axis_index("c")
    s = lax.axis_index("s")
    wid = s * NC + c
    b = wid // WPB
    t_base = (wid % WPB) * TOK_PER_W

    inbufs = (in0, in1)
    outbufs = (ob0, ob1)
    sems_i = (sem_i0, sem_i1)
    sems_o = (sem_o0, sem_o1)
    zv = jnp.zeros((16,), jnp.float32)

    pltpu.sync_copy(seg_hbm.at[pl.ds(b * L, L)], seg_v.at[pl.ds(0, L)])
    seg_v[pl.ds(L, 16)] = jnp.full((16,), L, jnp.int32)
    r_start = _lower_bound(seg_v, t_base)

    def in_dma(ci, a0, buf, sem):
        start = a0 + ci * RC
        d = jnp.minimum(start, L - RC)
        d = pl.multiple_of(d, 8)
        return pltpu.make_async_copy(x_hbm.at[b, pl.ds(d, RC), :], buf, sem)

    def out_dma(k, buf, sem):
        t_lo = pl.multiple_of(t_base + k * TT, TT)
        return pltpu.make_async_copy(buf, out_hbm.at[b, pl.ds(t_lo, TT), :], sem)

    def run_tile(kq, q, r0):
        k = 2 * kq + q
        t_lo = t_base + k * TT
        r1 = _lower_bound(seg_v, t_lo + TT)
        outbuf = outbufs[q]

        @pl.when(kq >= 1)
        def _():
            out_dma(k - 2, outbuf, sems_o[q]).wait()

        a0 = lax.bitwise_and(r0, -8)
        nch = (r1 - a0 + RC - 1) // RC

        @pl.when(nch > 0)
        def _():
            in_dma(0, a0, inbufs[0], sems_i[0]).start()

        def zero_tokens(t0, t1):
            def zbody(t, carry):
                for h in range(HV):
                    outbuf[t, pl.ds(16 * h, 16)] = zv
                return carry
            lax.fori_loop(t0, t1, zbody, 0)

        def chunk_pair(cj, carry):
            for p in (0, 1):
                ci = 2 * cj + p

                @pl.when(ci < nch)
                def _():
                    in_dma(ci, a0, inbufs[p], sems_i[p]).wait()

                    @pl.when(ci + 1 < nch)
                    def _():
                        in_dma(ci + 1, a0, inbufs[1 - p], sems_i[1 - p]).start()

                start = a0 + ci * RC
                d = jnp.minimum(start, L - RC)
                lo_i = jnp.maximum(r0, start)
                cnt = jnp.clip(r1 - lo_i, 0, start + RC - lo_i)
                inbuf = inbufs[p]

                def row_body(i, rc):
                    prev_s = rc[0]
                    acc = rc[1:]
                    pr = lo_i + i
                    sgid = _sget(seg_v, pr)
                    lr = pr - d
                    same = sgid == prev_s
                    zero_tokens(prev_s + 1 - t_lo, sgid - t_lo)
                    new_acc = []
                    for h in range(HV):
                        xv = inbuf[lr, pl.ds(16 * h, 16)]
                        new_acc.append(xv + jnp.where(same, acc[h], zv))
                    tl = sgid - t_lo
                    for h in range(HV):
                        outbuf[tl, pl.ds(16 * h, 16)] = new_acc[h]
                    return (sgid,) + tuple(new_acc)

                carry = lax.fori_loop(0, cnt, row_body, carry)
            return carry

        init = (t_lo - 1,) + tuple(zv for _ in range(HV))
        last = lax.fori_loop(0, (nch + 1) // 2, chunk_pair, init)
        zero_tokens(last[0] + 1 - t_lo, TT)
        out_dma(k, outbuf, sems_o[q]).start()
        return r1

    def tile_pair(kq, r0):
        r0 = run_tile(kq, 0, r0)
        r0 = run_tile(kq, 1, r0)
        return r0

    lax.fori_loop(0, NT // 2, tile_pair, r_start)
    out_dma(NT - 2, outbufs[0], sems_o[0]).wait()
    out_dma(NT - 1, outbufs[1], sems_o[1]).wait()


@jax.jit
def kernel(sequence_output, wp_segment_ids):
    seg = wp_segment_ids.astype(jnp.int32).reshape(B * L)
    run = pl.kernel(
        _body,
        out_type=jax.ShapeDtypeStruct((B, L, H), jnp.float32),
        mesh=plsc.VectorSubcoreMesh(core_axis_name="c", subcore_axis_name="s"),
        compiler_params=pltpu.CompilerParams(use_tc_tiling_on_sc=True),
        scratch_types=[
            pltpu.VMEM((L + 16,), jnp.int32),
            pltpu.VMEM((RC, H), jnp.float32),
            pltpu.VMEM((RC, H), jnp.float32),
            pltpu.VMEM((TT, H), jnp.float32),
            pltpu.VMEM((TT, H), jnp.float32),
            pltpu.SemaphoreType.DMA,
            pltpu.SemaphoreType.DMA,
            pltpu.SemaphoreType.DMA,
            pltpu.SemaphoreType.DMA,
        ],
    )
    return run(sequence_output, seg)
